# Initial kernel scaffold; baseline (speedup 1.0000x reference)
import jax
import jax.numpy as jnp
from jax.experimental import pallas as pl


def kernel(x, prob):
    raise NotImplementedError("write your pallas kernel here")


# speedup vs baseline: 1.0137x; 1.0137x over previous
import functools

import jax
import jax.numpy as jnp
from jax import lax
from jax.experimental import pallas as pl
from jax.experimental.pallas import tpu as pltpu
from jax.experimental.pallas import tpu_sc as plsc

_V = 1000
_NTOT = 1024 * 50
_NW = 32
_PER_W = _NTOT // _NW
_C = 64
_NCHUNK = _PER_W // _C


def _sc_gather(table, idx_flat):
  mesh = plsc.VectorSubcoreMesh(core_axis_name="c", subcore_axis_name="s")

  @functools.partial(
      pl.kernel,
      mesh=mesh,
      out_type=jax.ShapeDtypeStruct((_NTOT, _V), jnp.float32),
      compiler_params=pltpu.CompilerParams(use_tc_tiling_on_sc=False),
      scratch_types=[
          pltpu.VMEM((_PER_W,), jnp.int32),
          pltpu.VMEM((_C, _V), jnp.float32),
          pltpu.SemaphoreType.DMA,
      ],
  )
  def k(table_hbm, idx_hbm, out_hbm, idx_v, rows_v, sem):
    wid = lax.axis_index("s") * 2 + lax.axis_index("c")
    base = wid * _PER_W
    pltpu.sync_copy(idx_hbm.at[pl.ds(base, _PER_W)], idx_v)

    def body(g, carry):
      pltpu.async_copy(
          table_hbm.at[idx_v.at[pl.ds(g * _C, _C)]], rows_v, sem
      ).wait()
      pltpu.sync_copy(rows_v, out_hbm.at[pl.ds(base + g * _C, _C)])
      return carry

    lax.fori_loop(0, _NCHUNK, body, 0)

  return k(table, idx_flat)


def kernel(x, prob):
  idx = x.reshape(-1).astype(jnp.int32)
  out = _sc_gather(prob, idx)
  return out.reshape(x.shape[0], x.shape[1], _V)
